# baseline (speedup 1.0000x reference)
import functools

import jax
import jax.numpy as jnp
from jax import lax
from jax.experimental import pallas as pl
from jax.experimental.pallas import tpu as pltpu
from jax.experimental.pallas import tpu_sc as plsc

N_E = 8192
E_DIM = 32
BETA = 0.25
B = 16384
EPS = 1e-8


BT = 1024
NC = 2048


KPACK = 256 // E_DIM
NG = N_E // KPACK


NCH = NC // 128
RG = 64
NGB = NC // NG


def _fused_body(x_ref, w_ref, s_out, i_out, wn_out,
                wbig_s, xn_s, xm_s, vbest, cbest):
    c = pl.program_id(0)
    nc = pl.num_programs(0)
    t = pl.program_id(1)
    base = t * BT

    @pl.when(t == 0)
    def _():
        wb = w_ref[...]
        wn = wb / jnp.maximum(
            jnp.sqrt(jnp.sum(wb * wb, axis=-1, keepdims=True)), EPS)
        wn_out[...] = wn
        wbig_s[...] = jnp.zeros((256, NC), jnp.float32)
        for h in range(NGB):
            wbig_s[pl.ds(c * (E_DIM * NGB) + h * E_DIM, E_DIM),
                   h * NG:(h + 1) * NG] = lax.transpose(
                       wn[h * NG:(h + 1) * NG, :], (1, 0))

    @pl.when(c == 0)
    def _():
        xb = x_ref[...]
        xm = jnp.maximum(
            jnp.sqrt(jnp.sum(xb * xb, axis=-1, keepdims=True)), EPS)
        xn_s[pl.ds(base, BT), :] = xb / xm
        xm_s[pl.ds(base, BT)] = xm[:, 0]

    xn = xn_s[pl.ds(base, BT), :]
    xrep = jnp.concatenate([xn] * KPACK, axis=1)
    sim = lax.dot_general(xrep, wbig_s[...], (((1,), (0,)), ((), ())),
                          preferred_element_type=jnp.float32)

    for g in range(BT // RG):
        r0 = base + g * RG

        @pl.when(c == 0)
        def _(r0=r0, g=g):
            vb = sim[g * RG:(g + 1) * RG, 0:128]
            cb = jnp.zeros((RG, 128), jnp.int32)
            for k in range(1, NCH):
                s = sim[g * RG:(g + 1) * RG, k * 128:(k + 1) * 128]
                better = s > vb
                vb = jnp.where(better, s, vb)
                cb = jnp.where(better, jnp.int32(k), cb)
            vbest[pl.ds(r0, RG), :] = vb
            cbest[pl.ds(r0, RG), :] = cb

        @pl.when(c > 0)
        def _(r0=r0, g=g):
            vb = vbest[pl.ds(r0, RG), :]
            cb = cbest[pl.ds(r0, RG), :]
            for k in range(NCH):
                s = sim[g * RG:(g + 1) * RG, k * 128:(k + 1) * 128]
                better = s > vb
                vb = jnp.where(better, s, vb)
                cb = jnp.where(better, jnp.int32(c * NCH + k), cb)
            vbest[pl.ds(r0, RG), :] = vb
            cbest[pl.ds(r0, RG), :] = cb

    @pl.when(c == nc - 1)
    def _():
        vb = vbest[pl.ds(base, BT), :]
        cb = cbest[pl.ds(base, BT), :]
        m = jnp.max(vb, axis=1)
        lanes = lax.broadcasted_iota(jnp.int32, (BT, 128), 1)
        gidx = cb * 128 + lanes
        i_out[...] = jnp.min(
            jnp.where(vb == m[:, None], gidx, jnp.int32(2**30)), axis=1)
        s_out[...] = jnp.maximum(m * xm_s[pl.ds(base, BT)], 0.0)


def _tc_fused(x, W):
    grid = (N_E // NC, B // BT)
    return pl.pallas_call(
        _fused_body,
        grid=grid,
        in_specs=[
            pl.BlockSpec((BT, E_DIM), lambda c, t: (t, 0)),
            pl.BlockSpec((NC, E_DIM), lambda c, t: (c, 0)),
        ],
        out_specs=[
            pl.BlockSpec((BT,), lambda c, t: (t,)),
            pl.BlockSpec((BT,), lambda c, t: (t,)),
            pl.BlockSpec((NC, E_DIM), lambda c, t: (c, 0)),
        ],
        out_shape=[
            jax.ShapeDtypeStruct((B,), jnp.float32),
            jax.ShapeDtypeStruct((B,), jnp.int32),
            jax.ShapeDtypeStruct((N_E, E_DIM), jnp.float32),
        ],
        scratch_shapes=[
            pltpu.VMEM((256, NC), jnp.float32),
            pltpu.VMEM((B, E_DIM), jnp.float32),
            pltpu.VMEM((B,), jnp.float32),
            pltpu.VMEM((B, 128), jnp.float32),
            pltpu.VMEM((B, 128), jnp.int32),
        ],
    )(x, W)


_NCORES = 2
_NSUB = 16
NW = _NCORES * _NSUB
TPW = B // NW
GCHUNK = 128


def _sc_body(wn_hbm, idx_hbm, val_hbm, x_hbm,
             xq_hbm, part_hbm,
             idx_v, rows_v, val_v, x_v, xq_v, acc_v, sem):
    wid = lax.axis_index("s") * _NCORES + lax.axis_index("c")
    base = wid * TPW

    pltpu.sync_copy(idx_hbm.at[pl.ds(base, TPW)], idx_v)
    pltpu.sync_copy(val_hbm.at[pl.ds(base, TPW)], val_v)
    pltpu.sync_copy(x_hbm.at[pl.ds(base, TPW)], x_v)
    for j in range(TPW // GCHUNK):
        pltpu.async_copy(
            wn_hbm.at[idx_v.at[pl.ds(j * GCHUNK, GCHUNK)]],
            rows_v.at[pl.ds(j * GCHUNK, GCHUNK)],
            sem,
        ).wait()

    def body(g, acc):
        vv = val_v[pl.ds(g * 16, 16)]
        for i in range(16):
            t = g * 16 + i
            sv = jnp.full((16,), vv[i], jnp.float32)
            for h in range(E_DIM // 16):
                d = rows_v[t, pl.ds(16 * h, 16)]
                xv = x_v[t, pl.ds(16 * h, 16)]
                e = sv * d - xv
                xq_v[t, pl.ds(16 * h, 16)] = xv + e
                acc = acc + e * e
        return acc

    acc = lax.fori_loop(0, TPW // 16, body, jnp.zeros((16,), jnp.float32))
    acc_v[...] = acc
    pltpu.sync_copy(xq_v, xq_hbm.at[pl.ds(base, TPW)])
    pltpu.sync_copy(acc_v, part_hbm.at[wid])


def _sc_gather(wn, idx, val, x):
    mesh = plsc.VectorSubcoreMesh(core_axis_name="c", subcore_axis_name="s")
    k = functools.partial(
        pl.kernel,
        mesh=mesh,
        out_type=[
            jax.ShapeDtypeStruct((B, E_DIM), jnp.float32),
            jax.ShapeDtypeStruct((NW, 16), jnp.float32),
        ],
        scratch_types=[
            pltpu.VMEM((TPW,), jnp.int32),
            pltpu.VMEM((TPW, E_DIM), jnp.float32),
            pltpu.VMEM((TPW,), jnp.float32),
            pltpu.VMEM((TPW, E_DIM), jnp.float32),
            pltpu.VMEM((TPW, E_DIM), jnp.float32),
            pltpu.VMEM((16,), jnp.float32),
            pltpu.SemaphoreType.DMA,
        ],
        compiler_params=pltpu.CompilerParams(use_tc_tiling_on_sc=False),
    )(_sc_body)
    return k(wn, idx, val, x)


def kernel(x, W):
    scalar, indices, wn = _tc_fused(x, W)
    x_q, partials = _sc_gather(wn, indices, scalar, x)
    loss = BETA * (jnp.sum(partials) / jnp.float32(B * E_DIM))
    return (x_q, loss, indices, scalar)


# speedup vs baseline: 1.4441x; 1.4441x over previous
import functools

import jax
import jax.numpy as jnp
from jax import lax
from jax.experimental import pallas as pl
from jax.experimental.pallas import tpu as pltpu
from jax.experimental.pallas import tpu_sc as plsc

N_E = 8192
E_DIM = 32
BETA = 0.25
B = 16384
EPS = 1e-8


BT = 1024
NC = 2048


KPACK = 256 // E_DIM
NG = N_E // KPACK


NCH = NC // 128
RG = 64
NGB = NC // NG


def _fused_body(x_ref, w_ref, s_out, i_out, wn_out,
                wbig_s, xn_s, xm_s, vbest, cbest):
    c = pl.program_id(0)
    nc = pl.num_programs(0)
    t = pl.program_id(1)
    base = t * BT

    @pl.when(t == 0)
    def _():
        wb = w_ref[...]
        wn = wb / jnp.maximum(
            jnp.sqrt(jnp.sum(wb * wb, axis=-1, keepdims=True)), EPS)
        wn_out[...] = wn
        wbig_s[...] = jnp.zeros((256, NC), jnp.float32)
        for h in range(NGB):
            wbig_s[pl.ds(c * (E_DIM * NGB) + h * E_DIM, E_DIM),
                   h * NG:(h + 1) * NG] = lax.transpose(
                       wn[h * NG:(h + 1) * NG, :], (1, 0))

    @pl.when(c == 0)
    def _():
        xb = x_ref[...]
        xm = jnp.maximum(
            jnp.sqrt(jnp.sum(xb * xb, axis=-1, keepdims=True)), EPS)
        xn_s[pl.ds(base, BT), :] = xb / xm
        xm_s[pl.ds(base, BT)] = xm[:, 0]

    xn = xn_s[pl.ds(base, BT), :]
    xrep = jnp.concatenate([xn] * KPACK, axis=1)
    sim = lax.dot_general(xrep, wbig_s[...], (((1,), (0,)), ((), ())),
                          preferred_element_type=jnp.float32)

    first = c == 0
    for g in range(BT // RG):
        r0 = base + g * RG
        if_first = lambda a, b: jnp.where(first, a, b)
        vb = if_first(jnp.full((RG, 128), -jnp.inf, jnp.float32),
                      vbest[pl.ds(r0, RG), :])
        cb = if_first(jnp.zeros((RG, 128), jnp.int32),
                      cbest[pl.ds(r0, RG), :])
        for k in range(NCH):
            s = sim[g * RG:(g + 1) * RG, k * 128:(k + 1) * 128]
            better = s > vb
            vb = jnp.where(better, s, vb)
            cb = jnp.where(better, jnp.int32(c * NCH + k), cb)
        vbest[pl.ds(r0, RG), :] = vb
        cbest[pl.ds(r0, RG), :] = cb

    @pl.when(c == nc - 1)
    def _():
        vb = vbest[pl.ds(base, BT), :]
        cb = cbest[pl.ds(base, BT), :]
        m = jnp.max(vb, axis=1)
        lanes = lax.broadcasted_iota(jnp.int32, (BT, 128), 1)
        gidx = cb * 128 + lanes
        i_out[...] = jnp.min(
            jnp.where(vb == m[:, None], gidx, jnp.int32(2**30)), axis=1)
        s_out[...] = jnp.maximum(m * xm_s[pl.ds(base, BT)], 0.0)


def _tc_fused(x, W):
    grid = (N_E // NC, B // BT)
    return pl.pallas_call(
        _fused_body,
        grid=grid,
        in_specs=[
            pl.BlockSpec((BT, E_DIM), lambda c, t: (t, 0)),
            pl.BlockSpec((NC, E_DIM), lambda c, t: (c, 0)),
        ],
        out_specs=[
            pl.BlockSpec((BT,), lambda c, t: (t,)),
            pl.BlockSpec((BT,), lambda c, t: (t,)),
            pl.BlockSpec((NC, E_DIM), lambda c, t: (c, 0)),
        ],
        out_shape=[
            jax.ShapeDtypeStruct((B,), jnp.float32),
            jax.ShapeDtypeStruct((B,), jnp.int32),
            jax.ShapeDtypeStruct((N_E, E_DIM), jnp.float32),
        ],
        scratch_shapes=[
            pltpu.VMEM((256, NC), jnp.float32),
            pltpu.VMEM((B, E_DIM), jnp.float32),
            pltpu.VMEM((B,), jnp.float32),
            pltpu.VMEM((B, 128), jnp.float32),
            pltpu.VMEM((B, 128), jnp.int32),
        ],
    )(x, W)


_NCORES = 2
_NSUB = 16
NW = _NCORES * _NSUB
TPW = B // NW
GCHUNK = 128


def _sc_body(wn_hbm, idx_hbm, val_hbm, x_hbm,
             xq_hbm, part_hbm,
             idx_v, rows_v, val_v, x_v, xq_v, acc_v, sem):
    wid = lax.axis_index("s") * _NCORES + lax.axis_index("c")
    base = wid * TPW

    pltpu.sync_copy(idx_hbm.at[pl.ds(base, TPW)], idx_v)
    pltpu.sync_copy(val_hbm.at[pl.ds(base, TPW)], val_v)
    pltpu.sync_copy(x_hbm.at[pl.ds(base, TPW)], x_v)
    for j in range(TPW // GCHUNK):
        pltpu.async_copy(
            wn_hbm.at[idx_v.at[pl.ds(j * GCHUNK, GCHUNK)]],
            rows_v.at[pl.ds(j * GCHUNK, GCHUNK)],
            sem,
        ).wait()

    def body(g, acc):
        vv = val_v[pl.ds(g * 16, 16)]
        for i in range(16):
            t = g * 16 + i
            sv = jnp.full((16,), vv[i], jnp.float32)
            for h in range(E_DIM // 16):
                d = rows_v[t, pl.ds(16 * h, 16)]
                xv = x_v[t, pl.ds(16 * h, 16)]
                e = sv * d - xv
                xq_v[t, pl.ds(16 * h, 16)] = xv + e
                acc = acc + e * e
        return acc

    acc = lax.fori_loop(0, TPW // 16, body, jnp.zeros((16,), jnp.float32))
    acc_v[...] = acc
    pltpu.sync_copy(xq_v, xq_hbm.at[pl.ds(base, TPW)])
    pltpu.sync_copy(acc_v, part_hbm.at[wid])


def _sc_gather(wn, idx, val, x):
    mesh = plsc.VectorSubcoreMesh(core_axis_name="c", subcore_axis_name="s")
    k = functools.partial(
        pl.kernel,
        mesh=mesh,
        out_type=[
            jax.ShapeDtypeStruct((B, E_DIM), jnp.float32),
            jax.ShapeDtypeStruct((NW, 16), jnp.float32),
        ],
        scratch_types=[
            pltpu.VMEM((TPW,), jnp.int32),
            pltpu.VMEM((TPW, E_DIM), jnp.float32),
            pltpu.VMEM((TPW,), jnp.float32),
            pltpu.VMEM((TPW, E_DIM), jnp.float32),
            pltpu.VMEM((TPW, E_DIM), jnp.float32),
            pltpu.VMEM((16,), jnp.float32),
            pltpu.SemaphoreType.DMA,
        ],
        compiler_params=pltpu.CompilerParams(use_tc_tiling_on_sc=False),
    )(_sc_body)
    return k(wn, idx, val, x)


def kernel(x, W):
    scalar, indices, wn = _tc_fused(x, W)
    x_q, partials = _sc_gather(wn, indices, scalar, x)
    loss = BETA * (jnp.sum(partials) / jnp.float32(B * E_DIM))
    return (x_q, loss, indices, scalar)


# speedup vs baseline: 1.4619x; 1.0123x over previous
import functools

import jax
import jax.numpy as jnp
from jax import lax
from jax.experimental import pallas as pl
from jax.experimental.pallas import tpu as pltpu
from jax.experimental.pallas import tpu_sc as plsc

N_E = 8192
E_DIM = 32
BETA = 0.25
B = 16384
EPS = 1e-8


BT = 1024
NC = 2048


KPACK = 256 // E_DIM
NG = N_E // KPACK


NCH = NC // 128
RG = 64
NGB = NC // NG


def _fused_body(x_ref, w_ref, s_out, i_out, wn_out,
                wbig_s, xr_s, xm_s, vbest, cbest):
    c = pl.program_id(0)
    nc = pl.num_programs(0)
    t = pl.program_id(1)
    base = t * BT

    @pl.when(t == 0)
    def _():
        wb = w_ref[...]
        wn = wb / jnp.maximum(
            jnp.sqrt(jnp.sum(wb * wb, axis=-1, keepdims=True)), EPS)
        wn_out[...] = wn
        wbig_s[...] = jnp.zeros((256, NC), jnp.float32)
        for h in range(NGB):
            wbig_s[pl.ds(c * (E_DIM * NGB) + h * E_DIM, E_DIM),
                   h * NG:(h + 1) * NG] = lax.transpose(
                       wn[h * NG:(h + 1) * NG, :], (1, 0))

    @pl.when(c == 0)
    def _():
        xb = x_ref[...]
        xm = jnp.maximum(
            jnp.sqrt(jnp.sum(xb * xb, axis=-1, keepdims=True)), EPS)
        xn = xb / xm
        xr_s[pl.ds(base, BT), :] = jnp.concatenate([xn] * KPACK, axis=1)
        xm_s[pl.ds(base, BT)] = xm[:, 0]

    xrep = xr_s[pl.ds(base, BT), :]
    sim = lax.dot_general(xrep, wbig_s[...], (((1,), (0,)), ((), ())),
                          preferred_element_type=jnp.float32)

    first = c == 0
    for g in range(BT // RG):
        r0 = base + g * RG
        if_first = lambda a, b: jnp.where(first, a, b)
        vb = if_first(jnp.full((RG, 128), -jnp.inf, jnp.float32),
                      vbest[pl.ds(r0, RG), :])
        cb = if_first(jnp.zeros((RG, 128), jnp.int32),
                      cbest[pl.ds(r0, RG), :])
        for k in range(NCH):
            s = sim[g * RG:(g + 1) * RG, k * 128:(k + 1) * 128]
            better = s > vb
            vb = jnp.where(better, s, vb)
            cb = jnp.where(better, jnp.int32(c * NCH + k), cb)
        vbest[pl.ds(r0, RG), :] = vb
        cbest[pl.ds(r0, RG), :] = cb

    @pl.when(c == nc - 1)
    def _():
        vb = vbest[pl.ds(base, BT), :]
        cb = cbest[pl.ds(base, BT), :]
        m = jnp.max(vb, axis=1)
        lanes = lax.broadcasted_iota(jnp.int32, (BT, 128), 1)
        gidx = cb * 128 + lanes
        i_out[...] = jnp.min(
            jnp.where(vb == m[:, None], gidx, jnp.int32(2**30)), axis=1)
        s_out[...] = jnp.maximum(m * xm_s[pl.ds(base, BT)], 0.0)


def _tc_fused(x, W):
    grid = (N_E // NC, B // BT)
    return pl.pallas_call(
        _fused_body,
        grid=grid,
        in_specs=[
            pl.BlockSpec((BT, E_DIM), lambda c, t: (t, 0)),
            pl.BlockSpec((NC, E_DIM), lambda c, t: (c, 0)),
        ],
        out_specs=[
            pl.BlockSpec((BT,), lambda c, t: (t,)),
            pl.BlockSpec((BT,), lambda c, t: (t,)),
            pl.BlockSpec((NC, E_DIM), lambda c, t: (c, 0)),
        ],
        out_shape=[
            jax.ShapeDtypeStruct((B,), jnp.float32),
            jax.ShapeDtypeStruct((B,), jnp.int32),
            jax.ShapeDtypeStruct((N_E, E_DIM), jnp.float32),
        ],
        scratch_shapes=[
            pltpu.VMEM((256, NC), jnp.float32),
            pltpu.VMEM((B, KPACK * E_DIM), jnp.float32),
            pltpu.VMEM((B,), jnp.float32),
            pltpu.VMEM((B, 128), jnp.float32),
            pltpu.VMEM((B, 128), jnp.int32),
        ],
    )(x, W)


_NCORES = 2
_NSUB = 16
NW = _NCORES * _NSUB
TPW = B // NW
GCHUNK = 128


def _sc_body(wn_hbm, idx_hbm, val_hbm, x_hbm,
             xq_hbm, part_hbm,
             idx_v, rows_v, val_v, x_v, xq_v, acc_v, sem):
    wid = lax.axis_index("s") * _NCORES + lax.axis_index("c")
    base = wid * TPW

    pltpu.sync_copy(idx_hbm.at[pl.ds(base, TPW)], idx_v)
    pltpu.sync_copy(val_hbm.at[pl.ds(base, TPW)], val_v)
    pltpu.sync_copy(x_hbm.at[pl.ds(base, TPW)], x_v)
    for j in range(TPW // GCHUNK):
        pltpu.async_copy(
            wn_hbm.at[idx_v.at[pl.ds(j * GCHUNK, GCHUNK)]],
            rows_v.at[pl.ds(j * GCHUNK, GCHUNK)],
            sem,
        ).wait()

    def body(g, acc):
        vv = val_v[pl.ds(g * 16, 16)]
        for i in range(16):
            t = g * 16 + i
            sv = jnp.full((16,), vv[i], jnp.float32)
            for h in range(E_DIM // 16):
                d = rows_v[t, pl.ds(16 * h, 16)]
                xv = x_v[t, pl.ds(16 * h, 16)]
                e = sv * d - xv
                xq_v[t, pl.ds(16 * h, 16)] = xv + e
                acc = acc + e * e
        return acc

    acc = lax.fori_loop(0, TPW // 16, body, jnp.zeros((16,), jnp.float32))
    acc_v[...] = acc
    pltpu.sync_copy(xq_v, xq_hbm.at[pl.ds(base, TPW)])
    pltpu.sync_copy(acc_v, part_hbm.at[wid])


def _sc_gather(wn, idx, val, x):
    mesh = plsc.VectorSubcoreMesh(core_axis_name="c", subcore_axis_name="s")
    k = functools.partial(
        pl.kernel,
        mesh=mesh,
        out_type=[
            jax.ShapeDtypeStruct((B, E_DIM), jnp.float32),
            jax.ShapeDtypeStruct((NW, 16), jnp.float32),
        ],
        scratch_types=[
            pltpu.VMEM((TPW,), jnp.int32),
            pltpu.VMEM((TPW, E_DIM), jnp.float32),
            pltpu.VMEM((TPW,), jnp.float32),
            pltpu.VMEM((TPW, E_DIM), jnp.float32),
            pltpu.VMEM((TPW, E_DIM), jnp.float32),
            pltpu.VMEM((16,), jnp.float32),
            pltpu.SemaphoreType.DMA,
        ],
        compiler_params=pltpu.CompilerParams(use_tc_tiling_on_sc=False),
    )(_sc_body)
    return k(wn, idx, val, x)


def kernel(x, W):
    scalar, indices, wn = _tc_fused(x, W)
    x_q, partials = _sc_gather(wn, indices, scalar, x)
    loss = BETA * (jnp.sum(partials) / jnp.float32(B * E_DIM))
    return (x_q, loss, indices, scalar)
